# baseline (speedup 1.0000x reference)
import functools

import jax
import jax.numpy as jnp
from jax import lax
from jax.experimental import pallas as pl
from jax.experimental.pallas import tpu as pltpu
from jax.experimental.pallas import tpu_sc as plsc

_NUM_EXPERTS = 8
_RANK = 16
_SCALING = 2.0
_LORA_COLS = _NUM_EXPERTS * _RANK


def _logits_kernel(x_ref, r_ref, xb_ref, l_ref):
    x = x_ref[...]
    xb_ref[...] = x.astype(jnp.bfloat16)
    logits = jax.lax.dot_general(
        x, r_ref[...], (((1,), (1,)), ((), ())),
        preferred_element_type=jnp.float32)
    l_ref[...] = logits.T


def _routing_tc(logits):
    lane = jax.lax.broadcasted_iota(jnp.int32, logits.shape, 1)
    m1 = jnp.max(logits, axis=-1, keepdims=True)
    i1 = jnp.min(jnp.where(logits == m1, lane, _NUM_EXPERTS),
                 axis=-1, keepdims=True)
    l2 = jnp.where(lane == i1, -1e30, logits)
    m2 = jnp.max(l2, axis=-1, keepdims=True)
    i2 = jnp.min(jnp.where(l2 == m2, lane, _NUM_EXPERTS),
                 axis=-1, keepdims=True)
    e = jnp.exp(m2 - m1)
    inv = _SCALING / (1.0 + e)
    return jnp.where(lane == i1, inv, 0.0) + jnp.where(lane == i2, e * inv, 0.0)


def _make_sc_routing(n_tokens):
    info = plsc.get_sparse_core_info()
    chunk = 128
    nw = n_tokens // chunk
    mesh = plsc.VectorSubcoreMesh(core_axis_name="c", subcore_axis_name="s")

    @functools.partial(
        pl.kernel, mesh=mesh,
        out_type=jax.ShapeDtypeStruct((_NUM_EXPERTS, n_tokens), jnp.float32),
        scratch_types=[
            pltpu.VMEM((_NUM_EXPERTS, chunk), jnp.float32),
            pltpu.VMEM((_NUM_EXPERTS, chunk), jnp.float32),
        ],
    )
    def sc_routing(l_hbm, c_hbm, l_v, c_v):
        wid = lax.axis_index("s") * info.num_cores + lax.axis_index("c")
        base = wid * chunk

        @pl.when(wid < nw)
        def _work():
            _routing_chunk(l_hbm, c_hbm, l_v, c_v, base)

    def _routing_chunk(l_hbm, c_hbm, l_v, c_v, base):
        pltpu.sync_copy(l_hbm.at[:, pl.ds(base, chunk)], l_v)
        for g in range(chunk // 16):
            le = [l_v[e, pl.ds(g * 16, 16)] for e in range(_NUM_EXPERTS)]
            m1 = le[0]
            for e in range(1, _NUM_EXPERTS):
                m1 = jnp.maximum(m1, le[e])
            i1 = jnp.full((16,), _NUM_EXPERTS, jnp.int32)
            for e in range(_NUM_EXPERTS - 1, -1, -1):
                i1 = jnp.where(le[e] == m1, e, i1)
            l2 = [jnp.where(i1 == e, -1e30, le[e])
                  for e in range(_NUM_EXPERTS)]
            m2 = l2[0]
            for e in range(1, _NUM_EXPERTS):
                m2 = jnp.maximum(m2, l2[e])
            i2 = jnp.full((16,), _NUM_EXPERTS, jnp.int32)
            for e in range(_NUM_EXPERTS - 1, -1, -1):
                i2 = jnp.where(l2[e] == m2, e, i2)
            ex = jnp.exp(m2 - m1)
            inv = _SCALING / (1.0 + ex)
            c2 = ex * inv
            for e in range(_NUM_EXPERTS):
                ce = jnp.where(i1 == e, inv, jnp.where(i2 == e, c2, 0.0))
                c_v[e, pl.ds(g * 16, 16)] = ce
        pltpu.sync_copy(c_v, c_hbm.at[:, pl.ds(base, chunk)])

    return sc_routing


def _combine_kernel(xb_ref, w_ref, a_ref, b_ref, c_ref, o_ref, wa_ref):
    out_f = w_ref.shape[0]

    @pl.when(pl.program_id(0) == 0)
    def _cast_weights():
        wa_ref[:out_f, :] = w_ref[...].astype(jnp.bfloat16)
        wa_ref[out_f:, :] = a_ref[...].astype(jnp.bfloat16)

    xb = xb_ref[...]
    c = c_ref[...].T

    egrp = jax.lax.broadcasted_iota(
        jnp.int32, (xb.shape[0], _LORA_COLS), 1) // _RANK
    scale = jnp.where(egrp == 0, c[:, 0:1], 0.0)
    for e in range(1, _NUM_EXPERTS):
        scale = scale + jnp.where(egrp == e, c[:, e:e + 1], 0.0)

    y = jax.lax.dot_general(
        xb, wa_ref[...], (((1,), (1,)), ((), ())),
        preferred_element_type=jnp.float32)
    rs = (y[:, out_f:] * scale).astype(jnp.bfloat16)
    lora = jax.lax.dot_general(
        rs, b_ref[...], (((1,), (0,)), ((), ())),
        preferred_element_type=jnp.float32)
    o_ref[...] = y[:, :out_f] + lora


@functools.partial(jax.jit, static_argnames=("interpret",))
def kernel(x, weight, lora_A, lora_B, router_w, interpret=False):
    B, T, D = x.shape
    n = B * T
    out_f = weight.shape[0]
    x2 = x.reshape(n, D)
    a_all = lora_A.reshape(_LORA_COLS, D)
    b_all = lora_B.transpose(0, 2, 1).reshape(_LORA_COLS, out_f).astype(
        jnp.bfloat16)

    tm = 512
    grid = (n // tm,)

    xb, logits = pl.pallas_call(
        _logits_kernel,
        grid=grid,
        in_specs=[
            pl.BlockSpec((tm, D), lambda i: (i, 0)),
            pl.BlockSpec((_NUM_EXPERTS, D), lambda i: (0, 0)),
        ],
        out_specs=[
            pl.BlockSpec((tm, D), lambda i: (i, 0)),
            pl.BlockSpec((_NUM_EXPERTS, tm), lambda i: (0, i)),
        ],
        out_shape=[
            jax.ShapeDtypeStruct((n, D), jnp.bfloat16),
            jax.ShapeDtypeStruct((_NUM_EXPERTS, n), jnp.float32),
        ],
        interpret=interpret,
    )(x2, router_w)

    if interpret:
        c = _routing_tc(logits.T).T
    else:
        c = _make_sc_routing(n)(logits)

    out = pl.pallas_call(
        _combine_kernel,
        grid=grid,
        in_specs=[
            pl.BlockSpec((tm, D), lambda i: (i, 0)),
            pl.BlockSpec((out_f, D), lambda i: (0, 0)),
            pl.BlockSpec((_LORA_COLS, D), lambda i: (0, 0)),
            pl.BlockSpec((_LORA_COLS, out_f), lambda i: (0, 0)),
            pl.BlockSpec((_NUM_EXPERTS, tm), lambda i: (0, i)),
        ],
        out_specs=pl.BlockSpec((tm, out_f), lambda i: (i, 0)),
        out_shape=jax.ShapeDtypeStruct((n, out_f), jnp.float32),
        scratch_shapes=[pltpu.VMEM((out_f + _LORA_COLS, D), jnp.bfloat16)],
        interpret=interpret,
    )(xb, weight, a_all, b_all, c)
    return out.reshape(B, T, out_f)


# speedup vs baseline: 1.6058x; 1.6058x over previous
import jax
import jax.numpy as jnp
from jax.experimental import pallas as pl
from jax.experimental.pallas import tpu as pltpu

_NUM_EXPERTS = 8
_RANK = 16
_SCALING = 2.0
_LORA_COLS = _NUM_EXPERTS * _RANK


def _fused_kernel(x_ref, w_ref, a_ref, b_ref, r_ref, o_ref, wa_ref):
    out_f = w_ref.shape[0]

    @pl.when(pl.program_id(0) == 0)
    def _cast_weights():
        wa_ref[:out_f, :] = w_ref[...].astype(jnp.bfloat16)
        wa_ref[out_f:, :] = a_ref[...].astype(jnp.bfloat16)

    x = x_ref[...]
    xb = x.astype(jnp.bfloat16)

    logits = jax.lax.dot_general(
        x, r_ref[...], (((1,), (1,)), ((), ())),
        preferred_element_type=jnp.float32)
    lane = jax.lax.broadcasted_iota(jnp.int32, logits.shape, 1)
    m1 = jnp.max(logits, axis=-1, keepdims=True)
    i1 = jnp.min(jnp.where(logits == m1, lane, _NUM_EXPERTS),
                 axis=-1, keepdims=True)
    l2 = jnp.where(lane == i1, -1e30, logits)
    m2 = jnp.max(l2, axis=-1, keepdims=True)
    i2 = jnp.min(jnp.where(l2 == m2, lane, _NUM_EXPERTS),
                 axis=-1, keepdims=True)
    e = jnp.exp(m2 - m1)
    inv = _SCALING / (1.0 + e)
    c1 = inv
    c2 = e * inv

    egrp = jax.lax.broadcasted_iota(
        jnp.int32, (x.shape[0], _LORA_COLS), 1) // _RANK
    scale = jnp.where(egrp == i1, c1, 0.0) + jnp.where(egrp == i2, c2, 0.0)

    y = jax.lax.dot_general(
        xb, wa_ref[...], (((1,), (1,)), ((), ())),
        preferred_element_type=jnp.float32)
    rs = (y[:, out_f:] * scale).astype(jnp.bfloat16)
    lora = jax.lax.dot_general(
        rs, b_ref[...], (((1,), (0,)), ((), ())),
        preferred_element_type=jnp.float32)
    o_ref[...] = y[:, :out_f] + lora


@jax.jit
def kernel(x, weight, lora_A, lora_B, router_w):
    B, T, D = x.shape
    out_f = weight.shape[0]
    x2 = x.reshape(B * T, D)
    a_all = lora_A.reshape(_LORA_COLS, D)
    b_all = lora_B.transpose(0, 2, 1).reshape(_LORA_COLS, out_f).astype(
        jnp.bfloat16)

    tm = 512
    grid = (B * T // tm,)
    out = pl.pallas_call(
        _fused_kernel,
        grid=grid,
        in_specs=[
            pl.BlockSpec((tm, D), lambda i: (i, 0)),
            pl.BlockSpec((out_f, D), lambda i: (0, 0)),
            pl.BlockSpec((_LORA_COLS, D), lambda i: (0, 0)),
            pl.BlockSpec((_LORA_COLS, out_f), lambda i: (0, 0)),
            pl.BlockSpec((_NUM_EXPERTS, D), lambda i: (0, 0)),
        ],
        out_specs=pl.BlockSpec((tm, out_f), lambda i: (i, 0)),
        out_shape=jax.ShapeDtypeStruct((B * T, out_f), jnp.float32),
        scratch_shapes=[pltpu.VMEM((out_f + _LORA_COLS, D), jnp.bfloat16)],
    )(x2, weight, a_all, b_all, router_w)
    return out.reshape(B, T, out_f)
